# baseline (speedup 1.0000x reference)
import functools

import jax
import jax.numpy as jnp
from jax import lax
from jax.experimental import pallas as pl
from jax.experimental.pallas import tpu as pltpu
from jax.experimental.pallas import tpu_sc as plsc

N_USERS = 25000
N_ITEMS = 25000
N_NODES = N_USERS + N_ITEMS
EMB = 64
H = 32
E = 800000
BATCH = 4096
NB = 3 * BATCH
N_LAYERS = 3
DECAY = 1e-4

NC = 2
NS = 16
CHUNK = 128
CPB = 5
TILE_CHUNKS = 400
TILE_E = TILE_CHUNKS * CHUNK
E_PAD = NS * TILE_E
NROW2D = E_PAD // CHUNK
NBLK = TILE_CHUNKS // CPB
NPAIR = NBLK // 2

SIDE = 216
ROWS_PT = N_NODES // NS
SETUP_CHUNKS = 196

_mesh = plsc.VectorSubcoreMesh(
    core_axis_name="c", subcore_axis_name="s", num_cores=NC, num_subcores=NS)


def _sc_body(xraw, side, col2d, row2d, val2d, gidx, aidx, cidx, bidx,
             outsum, rawg,
             xflat, acc, stab, i1, i2, colv, rowv, valv, gbuf,
             semi, semg, sems, semd):
  c = lax.axis_index("c")
  s = lax.axis_index("s")
  half = c * (4 * N_NODES)
  rawb = c * N_NODES
  sideb = c * SIDE

  xraw_c = xraw.at[pl.ds(rawb, N_NODES)]
  z16 = jnp.zeros((16,), jnp.float32)

  def _zero_chunk(k):
    @plsc.parallel_loop(0, CHUNK)
    def _(r, k=k):
      gbuf[k, r, pl.ds(0, 16)] = z16
      gbuf[k, r, pl.ds(16, 16)] = z16

  def _fire_acc_zero():
    def body(j, _):
      r0 = s * ROWS_PT + jnp.minimum(j * CHUNK, ROWS_PT - CHUNK)
      pltpu.async_copy(gbuf.at[2], acc.at[pl.ds(r0, CHUNK)], semd.at[2])
      return 0
    lax.fori_loop(0, 25, body, 0)

  def _wait_acc_zero():
    def body(j, _):
      pltpu.make_async_copy(gbuf.at[2], acc.at[pl.ds(0, CHUNK)],
                            semd.at[2]).wait()
      return 0
    lax.fori_loop(0, 25, body, 0)

  _scope_setup = jax.named_scope("ph_setup"); _scope_setup.__enter__()
  pltpu.sync_copy(side.at[pl.ds(sideb, 13)], stab)
  cat_c = side.at[pl.ds(sideb + 13, 200)]
  _zero_chunk(2)
  _fire_acc_zero()

  def _setup_loop(base, total_rows, add_rows, combine):
    def body(j, _):
      cid = s + NS * j

      @pl.when(cid < SETUP_CHUNKS)
      def _():
        b = jnp.minimum(cid * CHUNK, total_rows - CHUNK)

        @pl.when(j > 0)
        def _():
          pltpu.make_async_copy(gbuf.at[0], xflat.at[pl.ds(0, CHUNK)],
                                semd.at[0]).wait()
        pltpu.async_copy(xraw_c.at[pl.ds(base + b, CHUNK)], gbuf.at[0],
                         semg.at[0])
        add_rows(b)
        pltpu.make_async_copy(xraw_c.at[pl.ds(0, CHUNK)], gbuf.at[0],
                              semg.at[0]).wait()
        combine()
        pltpu.async_copy(gbuf.at[0], xflat.at[pl.ds(half + base + b, CHUNK)],
                         semd.at[0])
      return 0

    lax.fori_loop(0, 13, body, 0)
    pltpu.make_async_copy(gbuf.at[0], xflat.at[pl.ds(0, CHUNK)],
                          semd.at[0]).wait()

  def _user_idx(b):
    pltpu.async_copy(gidx.at[pl.ds(b, CHUNK)], i1, semi.at[0])
    pltpu.async_copy(aidx.at[pl.ds(b, CHUNK)], i2, semi.at[1])
    pltpu.make_async_copy(gidx.at[pl.ds(0, CHUNK)], i1, semi.at[0]).wait()
    pltpu.make_async_copy(aidx.at[pl.ds(0, CHUNK)], i2, semi.at[1]).wait()

  def _combine_user():
    @plsc.parallel_loop(0, CHUNK // 16)
    def _(g):
      gv = i1[pl.ds(g * 16, 16)]
      av = i2[pl.ds(g * 16, 16)]
      for i16 in range(16):
        e = g * 16 + i16
        gi = gv[i16]
        ai = av[i16] + 3
        for h in range(2):
          sl = pl.ds(h * 16, 16)
          gbuf[0, e, sl] = gbuf[0, e, sl] + stab[gi, sl] + stab[ai, sl]

  def _item_idx(b):
    pltpu.async_copy(cidx.at[pl.ds(b, CHUNK)], i1, semi.at[0])
    pltpu.make_async_copy(cidx.at[pl.ds(0, CHUNK)], i1, semi.at[0]).wait()
    pltpu.async_copy(cat_c.at[i1], gbuf.at[3], semg.at[2])

  def _combine_item():
    pltpu.make_async_copy(cat_c.at[i1], gbuf.at[3], semg.at[2]).wait()

    @plsc.parallel_loop(0, CHUNK)
    def _(r):
      for h in range(2):
        sl = pl.ds(h * 16, 16)
        gbuf[0, r, sl] = gbuf[0, r, sl] + gbuf[3, r, sl]

  _setup_loop(0, N_USERS, _user_idx, _combine_user)
  _setup_loop(N_USERS, N_ITEMS, _item_idx, _combine_item)

  _wait_acc_zero()
  plsc.subcore_barrier()
  _scope_setup.__exit__(None, None, None)

  def _fire_idx(buf, b):
    cr = s * TILE_CHUNKS + b * CPB
    pltpu.async_copy(col2d.at[pl.ds(cr, CPB)], colv.at[buf], semi.at[buf])
    pltpu.async_copy(row2d.at[pl.ds(cr, CPB)], rowv.at[buf], semi.at[buf])
    pltpu.async_copy(val2d.at[pl.ds(cr, CPB)], valv.at[buf], semi.at[buf])

  def _wait_idx(buf, b):
    cr = s * TILE_CHUNKS + b * CPB
    pltpu.make_async_copy(col2d.at[pl.ds(cr, CPB)], colv.at[buf],
                          semi.at[buf]).wait()
    pltpu.make_async_copy(row2d.at[pl.ds(cr, CPB)], rowv.at[buf],
                          semi.at[buf]).wait()
    pltpu.make_async_copy(val2d.at[pl.ds(cr, CPB)], valv.at[buf],
                          semi.at[buf]).wait()

  for l in range(N_LAYERS):
    _scope_l = jax.named_scope(f"ph_layer{l}"); _scope_l.__enter__()
    src = half + l * N_NODES
    dst = half + (l + 1) * N_NODES
    xsrc = xflat.at[pl.ds(src, N_NODES)]

    _fire_idx(0, 0)

    def pair_body(i, _, xsrc=xsrc):
      for p in range(2):
        b = 2 * i + p
        q = 1 - p
        _wait_idx(p, b)
        for j in range(CPB):
          @pl.when(b > 0)
          def _(j=j, q=q):
            pltpu.make_async_copy(gbuf.at[j], acc.at[rowv.at[q, j]],
                                  sems.at[j]).wait()
          pltpu.async_copy(xsrc.at[colv.at[p, j]], gbuf.at[j], semg.at[j])
        @pl.when(b + 1 < NBLK)
        def _(q=q, b=b):
          _fire_idx(q, b + 1)
        for j in range(CPB):
          pltpu.make_async_copy(xsrc.at[colv.at[p, j]], gbuf.at[j],
                                semg.at[j]).wait()

          @plsc.parallel_loop(0, CHUNK // 16)
          def _(g, j=j, p=p):
            vv = valv[p, j, pl.ds(g * 16, 16)]
            for i16 in range(16):
              e = g * 16 + i16
              v = vv[i16]
              for h in range(2):
                sl = pl.ds(h * 16, 16)
                gbuf[j, e, sl] = gbuf[j, e, sl] * v

          pltpu.async_copy(gbuf.at[j], acc.at[rowv.at[p, j]], sems.at[j],
                           add=True)
      return 0

    lax.fori_loop(0, NPAIR, pair_body, 0)
    for j in range(CPB):
      pltpu.make_async_copy(gbuf.at[j], acc.at[rowv.at[1, j]],
                            sems.at[j]).wait()
    plsc.subcore_barrier()
    def dump_fire(j, _, dst=dst):
      r0 = s * ROWS_PT + jnp.minimum(j * CHUNK, ROWS_PT - CHUNK)
      pltpu.async_copy(acc.at[pl.ds(r0, CHUNK)],
                       xflat.at[pl.ds(dst + r0, CHUNK)], semd.at[2])
      return 0

    def dump_wait(j, _):
      pltpu.make_async_copy(acc.at[pl.ds(0, CHUNK)], xflat.at[pl.ds(0, CHUNK)],
                            semd.at[2]).wait()
      return 0

    lax.fori_loop(0, 25, dump_fire, 0)
    _zero_chunk(2)
    lax.fori_loop(0, 25, dump_wait, 0)
    _fire_acc_zero()
    _wait_acc_zero()
    plsc.subcore_barrier()
    _scope_l.__exit__(None, None, None)

  _scope_f = jax.named_scope("ph_final"); _scope_f.__enter__()
  ob = c * NB

  def fbody(j, _):
    cid = s * 6 + j
    bb = cid * CHUNK
    pltpu.async_copy(bidx.at[pl.ds(bb, CHUNK)], i1, semi.at[0])
    pltpu.make_async_copy(bidx.at[pl.ds(0, CHUNK)], i1, semi.at[0]).wait()

    @pl.when(j > 0)
    def _():
      pltpu.make_async_copy(gbuf.at[0], outsum.at[pl.ds(0, CHUNK)],
                            semd.at[0]).wait()
      pltpu.make_async_copy(gbuf.at[1], outsum.at[pl.ds(0, CHUNK)],
                            semd.at[1]).wait()

    for l in range(N_LAYERS + 1):
      xl = xflat.at[pl.ds(half + l * N_NODES, N_NODES)]
      pltpu.async_copy(xl.at[i1], gbuf.at[l], semg.at[l])
    for l in range(N_LAYERS + 1):
      pltpu.make_async_copy(xflat.at[pl.ds(0, N_NODES)].at[i1], gbuf.at[l],
                            semg.at[l]).wait()

    @plsc.parallel_loop(0, CHUNK)
    def _(r):
      for h in range(2):
        sl = pl.ds(h * 16, 16)
        gbuf[0, r, sl] = ((gbuf[0, r, sl] + gbuf[1, r, sl]) +
                          (gbuf[2, r, sl] + gbuf[3, r, sl]))

    pltpu.async_copy(xraw_c.at[i1], gbuf.at[1], semg.at[1])
    pltpu.async_copy(gbuf.at[0], outsum.at[pl.ds(ob + bb, CHUNK)], semd.at[0])
    pltpu.make_async_copy(xraw_c.at[i1], gbuf.at[1], semg.at[1]).wait()
    pltpu.async_copy(gbuf.at[1], rawg.at[pl.ds(ob + bb, CHUNK)], semd.at[1])
    return 0

  lax.fori_loop(0, 6, fbody, 0)
  for eb in range(2):
    pltpu.make_async_copy(gbuf.at[eb], outsum.at[pl.ds(0, CHUNK)],
                          semd.at[eb]).wait()
    _scope_f.__exit__(None, None, None)


_sc_prop = functools.partial(
    pl.kernel,
    out_type=[
        jax.ShapeDtypeStruct((NC * NB, H), jnp.float32),
        jax.ShapeDtypeStruct((NC * NB, H), jnp.float32),
    ],
    mesh=_mesh,
    compiler_params=pltpu.CompilerParams(use_tc_tiling_on_sc=False),
    scratch_types=[
        pltpu.HBM((NC * 4 * N_NODES, H), jnp.float32),
        pltpu.VMEM_SHARED((N_NODES, H), jnp.float32),
        pltpu.VMEM((13, H), jnp.float32),
        pltpu.VMEM((CHUNK,), jnp.int32),
        pltpu.VMEM((CHUNK,), jnp.int32),
        pltpu.VMEM((2, CPB, CHUNK), jnp.int32),
        pltpu.VMEM((2, CPB, CHUNK), jnp.int32),
        pltpu.VMEM((2, CPB, CHUNK), jnp.float32),
        pltpu.VMEM((CPB, CHUNK, H), jnp.float32),
        pltpu.SemaphoreType.DMA((2,)),
        pltpu.SemaphoreType.DMA((CPB,)),
        pltpu.SemaphoreType.DMA((CPB,)),
        pltpu.SemaphoreType.DMA((3,)),
    ],
)(_sc_body)


def _loss_body(u_ref, p_ref, n_ref, raw_ref, loss_ref, bpr_ref):
  u = u_ref[...]
  p = p_ref[...]
  n = n_ref[...]
  pos = jnp.sum(u * p, axis=1)
  neg = jnp.sum(u * n, axis=1)
  diff = (pos - neg) * (1.0 / 16.0)
  bpr = -jnp.mean(jax.nn.log_sigmoid(diff))
  raw = raw_ref[...]
  reg = jnp.sum(raw * raw) * (1.0 / BATCH)
  loss_ref[...] = jnp.reshape(bpr + DECAY * reg, (1, 1))
  bpr_ref[...] = jnp.reshape(bpr, (1, 1))


def kernel(edge_index, edge_values, user_gender, user_age_bucket, item_cat,
           users, pos_items, neg_items,
           user_emb, item_emb, gender_emb, age_emb, cat_emb):
  f32, i32 = jnp.float32, jnp.int32
  pad = E_PAD - E
  rowp = jnp.concatenate([edge_index[0], jnp.zeros((pad,), i32)])
  colp = jnp.concatenate([edge_index[1], jnp.zeros((pad,), i32)])
  valp = jnp.concatenate([edge_values, jnp.zeros((pad,), f32)])
  rowp = rowp.reshape(NROW2D, CHUNK)
  colp = colp.reshape(NROW2D, CHUNK)
  valp = valp.reshape(NROW2D, CHUNK)

  raw_lo = jnp.concatenate([user_emb[:, :H], item_emb[:, :H]], axis=0)
  raw_hi = jnp.concatenate([user_emb[:, H:], item_emb[:, H:]], axis=0)
  xraw = jnp.concatenate([raw_lo, raw_hi], axis=0)

  side_full = jnp.concatenate(
      [gender_emb, age_emb, cat_emb, jnp.zeros((3, EMB), f32)], axis=0)
  side = jnp.concatenate([side_full[:, :H], side_full[:, H:]], axis=0)

  bidx = jnp.concatenate(
      [users, N_USERS + pos_items, N_USERS + neg_items]).astype(i32)

  outsum, rawg = _sc_prop(xraw, side, colp, rowp, valp,
                          user_gender, user_age_bucket, item_cat, bidx)

  osum = jnp.concatenate([outsum[:NB], outsum[NB:]], axis=1)
  raw = jnp.concatenate([rawg[:NB], rawg[NB:]], axis=1)
  u = osum[:BATCH]
  p = osum[BATCH:2 * BATCH]
  n = osum[2 * BATCH:]

  loss, bpr = pl.pallas_call(
      _loss_body,
      out_shape=[jax.ShapeDtypeStruct((1, 1), f32)] * 2,
  )(u, p, n, raw)
  return (loss[0, 0], bpr[0, 0])


# speedup vs baseline: 1.6618x; 1.6618x over previous
import functools

import jax
import jax.numpy as jnp
from jax import lax
from jax.experimental import pallas as pl
from jax.experimental.pallas import tpu as pltpu
from jax.experimental.pallas import tpu_sc as plsc

N_USERS = 25000
N_ITEMS = 25000
N_NODES = N_USERS + N_ITEMS
EMB = 64
H = 32
E = 800000
BATCH = 4096
NB = 3 * BATCH
N_LAYERS = 3
DECAY = 1e-4

NC = 2
NS = 16
CHUNK = 128
CPB = 4
TILE_CHUNKS = 392
TILE_E = TILE_CHUNKS * CHUNK
E_PAD = NS * TILE_E
NROW2D = E_PAD // CHUNK
NBLK = TILE_CHUNKS // CPB
NPAIR = NBLK // 2

SIDE = 216
ROWS_PT = N_NODES // NS
SETUP_CHUNKS = 196

_mesh = plsc.VectorSubcoreMesh(
    core_axis_name="c", subcore_axis_name="s", num_cores=NC, num_subcores=NS)


def _sc_body(xraw, side, col2d, row2d, val2d, gidx, aidx, cidx, bidx,
             outsum, rawg,
             xflat, acc, stab, i1, i2, colv, rowv, valv, gbuf,
             semi, semg, sems, semd):
  c = lax.axis_index("c")
  s = lax.axis_index("s")
  half = c * (4 * N_NODES)
  rawb = c * N_NODES
  sideb = c * SIDE

  xraw_c = xraw.at[pl.ds(rawb, N_NODES)]
  z16 = jnp.zeros((16,), jnp.float32)

  def _zero_chunk(k):
    @plsc.parallel_loop(0, CHUNK)
    def _(r, k=k):
      gbuf[k, r, pl.ds(0, 16)] = z16
      gbuf[k, r, pl.ds(16, 16)] = z16

  def _fire_acc_zero():
    def body(j, _):
      r0 = s * ROWS_PT + jnp.minimum(j * CHUNK, ROWS_PT - CHUNK)
      pltpu.async_copy(gbuf.at[2], acc.at[pl.ds(r0, CHUNK)], semd.at[2])
      return 0
    lax.fori_loop(0, 25, body, 0)

  def _wait_acc_zero():
    def body(j, _):
      pltpu.make_async_copy(gbuf.at[2], acc.at[pl.ds(0, CHUNK)],
                            semd.at[2]).wait()
      return 0
    lax.fori_loop(0, 25, body, 0)

  _scope_setup = jax.named_scope("ph_setup"); _scope_setup.__enter__()
  pltpu.sync_copy(side.at[pl.ds(sideb, 13)], stab)
  cat_c = side.at[pl.ds(sideb + 13, 200)]
  _zero_chunk(2)
  _fire_acc_zero()

  def _setup_loop(base, total_rows, add_rows, combine):
    def body(j, _):
      cid = s + NS * j

      @pl.when(cid < SETUP_CHUNKS)
      def _():
        b = jnp.minimum(cid * CHUNK, total_rows - CHUNK)

        @pl.when(j > 0)
        def _():
          pltpu.make_async_copy(gbuf.at[0], xflat.at[pl.ds(0, CHUNK)],
                                semd.at[0]).wait()
        pltpu.async_copy(xraw_c.at[pl.ds(base + b, CHUNK)], gbuf.at[0],
                         semg.at[0])
        add_rows(b)
        pltpu.make_async_copy(xraw_c.at[pl.ds(0, CHUNK)], gbuf.at[0],
                              semg.at[0]).wait()
        combine()
        pltpu.async_copy(gbuf.at[0], xflat.at[pl.ds(half + base + b, CHUNK)],
                         semd.at[0])
      return 0

    lax.fori_loop(0, 13, body, 0)
    pltpu.make_async_copy(gbuf.at[0], xflat.at[pl.ds(0, CHUNK)],
                          semd.at[0]).wait()

  def _user_idx(b):
    pltpu.async_copy(gidx.at[pl.ds(b, CHUNK)], i1, semi.at[0])
    pltpu.async_copy(aidx.at[pl.ds(b, CHUNK)], i2, semi.at[1])
    pltpu.make_async_copy(gidx.at[pl.ds(0, CHUNK)], i1, semi.at[0]).wait()
    pltpu.make_async_copy(aidx.at[pl.ds(0, CHUNK)], i2, semi.at[1]).wait()

  def _combine_user():
    @plsc.parallel_loop(0, CHUNK // 16)
    def _(g):
      gv = i1[pl.ds(g * 16, 16)]
      av = i2[pl.ds(g * 16, 16)]
      for i16 in range(16):
        e = g * 16 + i16
        gi = gv[i16]
        ai = av[i16] + 3
        for h in range(2):
          sl = pl.ds(h * 16, 16)
          gbuf[0, e, sl] = gbuf[0, e, sl] + stab[gi, sl] + stab[ai, sl]

  def _item_idx(b):
    pltpu.async_copy(cidx.at[pl.ds(b, CHUNK)], i1, semi.at[0])
    pltpu.make_async_copy(cidx.at[pl.ds(0, CHUNK)], i1, semi.at[0]).wait()
    pltpu.async_copy(cat_c.at[i1], gbuf.at[3], semg.at[2])

  def _combine_item():
    pltpu.make_async_copy(cat_c.at[i1], gbuf.at[3], semg.at[2]).wait()

    @plsc.parallel_loop(0, CHUNK)
    def _(r):
      for h in range(2):
        sl = pl.ds(h * 16, 16)
        gbuf[0, r, sl] = gbuf[0, r, sl] + gbuf[3, r, sl]

  _setup_loop(0, N_USERS, _user_idx, _combine_user)
  _setup_loop(N_USERS, N_ITEMS, _item_idx, _combine_item)

  _wait_acc_zero()
  plsc.subcore_barrier()
  _scope_setup.__exit__(None, None, None)

  def _fire_idx(buf, b):
    cr = s * TILE_CHUNKS + b * CPB
    pltpu.async_copy(col2d.at[pl.ds(cr, CPB)], colv.at[buf], semi.at[buf])
    pltpu.async_copy(row2d.at[pl.ds(cr, CPB)], rowv.at[buf], semi.at[buf])
    pltpu.async_copy(val2d.at[pl.ds(cr, CPB)], valv.at[buf], semi.at[buf])

  def _wait_idx(buf, b):
    cr = s * TILE_CHUNKS + b * CPB
    pltpu.make_async_copy(col2d.at[pl.ds(cr, CPB)], colv.at[buf],
                          semi.at[buf]).wait()
    pltpu.make_async_copy(row2d.at[pl.ds(cr, CPB)], rowv.at[buf],
                          semi.at[buf]).wait()
    pltpu.make_async_copy(val2d.at[pl.ds(cr, CPB)], valv.at[buf],
                          semi.at[buf]).wait()

  for l in range(N_LAYERS):
    _scope_l = jax.named_scope(f"ph_layer{l}"); _scope_l.__enter__()
    src = half + l * N_NODES
    dst = half + (l + 1) * N_NODES
    xsrc = xflat.at[pl.ds(src, N_NODES)]

    _fire_idx(0, 0)

    def pair_body(i, _, xsrc=xsrc):
      for p in range(2):
        b = 2 * i + p
        q = 1 - p
        _wait_idx(p, b)
        for j in range(CPB):
          @pl.when(b > 0)
          def _(j=j, q=q):
            pltpu.make_async_copy(gbuf.at[j], acc.at[rowv.at[q, j]],
                                  sems.at[j]).wait()
          pltpu.async_copy(xsrc.at[colv.at[p, j]], gbuf.at[j], semg.at[j])
        @pl.when(b + 1 < NBLK)
        def _(q=q, b=b):
          _fire_idx(q, b + 1)
        for j in range(CPB):
          pltpu.make_async_copy(xsrc.at[colv.at[p, j]], gbuf.at[j],
                                semg.at[j]).wait()

          @plsc.parallel_loop(0, CHUNK // 16)
          def _(g, j=j, p=p):
            vv = valv[p, j, pl.ds(g * 16, 16)]
            for i16 in range(16):
              e = g * 16 + i16
              v = vv[i16]
              for h in range(2):
                sl = pl.ds(h * 16, 16)
                gbuf[j, e, sl] = gbuf[j, e, sl] * v

          pltpu.async_copy(gbuf.at[j], acc.at[rowv.at[p, j]], sems.at[j],
                           add=True)
      return 0

    lax.fori_loop(0, NPAIR, pair_body, 0)
    for j in range(CPB):
      pltpu.make_async_copy(gbuf.at[j], acc.at[rowv.at[1, j]],
                            sems.at[j]).wait()
    plsc.subcore_barrier()
    def dump_fire(j, _, dst=dst):
      r0 = s * ROWS_PT + jnp.minimum(j * CHUNK, ROWS_PT - CHUNK)
      pltpu.async_copy(acc.at[pl.ds(r0, CHUNK)],
                       xflat.at[pl.ds(dst + r0, CHUNK)], semd.at[2])
      return 0

    def dump_wait(j, _):
      pltpu.make_async_copy(acc.at[pl.ds(0, CHUNK)], xflat.at[pl.ds(0, CHUNK)],
                            semd.at[2]).wait()
      return 0

    lax.fori_loop(0, 25, dump_fire, 0)
    _zero_chunk(2)
    lax.fori_loop(0, 25, dump_wait, 0)
    _fire_acc_zero()
    _wait_acc_zero()
    plsc.subcore_barrier()
    _scope_l.__exit__(None, None, None)

  _scope_f = jax.named_scope("ph_final"); _scope_f.__enter__()
  ob = c * NB

  def fbody(j, _):
    cid = s * 6 + j
    bb = cid * CHUNK
    pltpu.async_copy(bidx.at[pl.ds(bb, CHUNK)], i1, semi.at[0])
    pltpu.make_async_copy(bidx.at[pl.ds(0, CHUNK)], i1, semi.at[0]).wait()

    @pl.when(j > 0)
    def _():
      pltpu.make_async_copy(gbuf.at[0], outsum.at[pl.ds(0, CHUNK)],
                            semd.at[0]).wait()
      pltpu.make_async_copy(gbuf.at[1], outsum.at[pl.ds(0, CHUNK)],
                            semd.at[1]).wait()

    for l in range(N_LAYERS + 1):
      xl = xflat.at[pl.ds(half + l * N_NODES, N_NODES)]
      pltpu.async_copy(xl.at[i1], gbuf.at[l], semg.at[l])
    for l in range(N_LAYERS + 1):
      pltpu.make_async_copy(xflat.at[pl.ds(0, N_NODES)].at[i1], gbuf.at[l],
                            semg.at[l]).wait()

    @plsc.parallel_loop(0, CHUNK)
    def _(r):
      for h in range(2):
        sl = pl.ds(h * 16, 16)
        gbuf[0, r, sl] = ((gbuf[0, r, sl] + gbuf[1, r, sl]) +
                          (gbuf[2, r, sl] + gbuf[3, r, sl]))

    pltpu.async_copy(xraw_c.at[i1], gbuf.at[1], semg.at[1])
    pltpu.async_copy(gbuf.at[0], outsum.at[pl.ds(ob + bb, CHUNK)], semd.at[0])
    pltpu.make_async_copy(xraw_c.at[i1], gbuf.at[1], semg.at[1]).wait()
    pltpu.async_copy(gbuf.at[1], rawg.at[pl.ds(ob + bb, CHUNK)], semd.at[1])
    return 0

  lax.fori_loop(0, 6, fbody, 0)
  for eb in range(2):
    pltpu.make_async_copy(gbuf.at[eb], outsum.at[pl.ds(0, CHUNK)],
                          semd.at[eb]).wait()
    _scope_f.__exit__(None, None, None)


_sc_prop = functools.partial(
    pl.kernel,
    out_type=[
        jax.ShapeDtypeStruct((NC * NB, H), jnp.float32),
        jax.ShapeDtypeStruct((NC * NB, H), jnp.float32),
    ],
    mesh=_mesh,
    compiler_params=pltpu.CompilerParams(use_tc_tiling_on_sc=False),
    scratch_types=[
        pltpu.HBM((NC * 4 * N_NODES, H), jnp.float32),
        pltpu.VMEM_SHARED((N_NODES, H), jnp.float32),
        pltpu.VMEM((13, H), jnp.float32),
        pltpu.VMEM((CHUNK,), jnp.int32),
        pltpu.VMEM((CHUNK,), jnp.int32),
        pltpu.VMEM((2, CPB, CHUNK), jnp.int32),
        pltpu.VMEM((2, CPB, CHUNK), jnp.int32),
        pltpu.VMEM((2, CPB, CHUNK), jnp.float32),
        pltpu.VMEM((CPB, CHUNK, H), jnp.float32),
        pltpu.SemaphoreType.DMA((2,)),
        pltpu.SemaphoreType.DMA((CPB,)),
        pltpu.SemaphoreType.DMA((CPB,)),
        pltpu.SemaphoreType.DMA((3,)),
    ],
)(_sc_body)


def _loss_body(out_ref, raw_ref, loss_ref, bpr_ref):
  o = out_ref[...]
  pos = jnp.zeros((BATCH,), jnp.float32)
  neg = jnp.zeros((BATCH,), jnp.float32)
  for hb in range(2):
    u = o[hb * NB:hb * NB + BATCH]
    p = o[hb * NB + BATCH:hb * NB + 2 * BATCH]
    n = o[hb * NB + 2 * BATCH:hb * NB + 3 * BATCH]
    pos = pos + jnp.sum(u * p, axis=1)
    neg = neg + jnp.sum(u * n, axis=1)
  diff = (pos - neg) * (1.0 / 16.0)
  bpr = -jnp.mean(jax.nn.log_sigmoid(diff))
  raw = raw_ref[...]
  reg = jnp.sum(raw * raw) * (1.0 / BATCH)
  loss_ref[...] = jnp.reshape(bpr + DECAY * reg, (1, 1))
  bpr_ref[...] = jnp.reshape(bpr, (1, 1))


def kernel(edge_index, edge_values, user_gender, user_age_bucket, item_cat,
           users, pos_items, neg_items,
           user_emb, item_emb, gender_emb, age_emb, cat_emb):
  f32, i32 = jnp.float32, jnp.int32
  pad = E_PAD - E
  rowp = jnp.concatenate([edge_index[0], jnp.zeros((pad,), i32)])
  colp = jnp.concatenate([edge_index[1], jnp.zeros((pad,), i32)])
  valp = jnp.concatenate([edge_values, jnp.zeros((pad,), f32)])
  rowp = rowp.reshape(NROW2D, CHUNK)
  colp = colp.reshape(NROW2D, CHUNK)
  valp = valp.reshape(NROW2D, CHUNK)

  raw_lo = jnp.concatenate([user_emb[:, :H], item_emb[:, :H]], axis=0)
  raw_hi = jnp.concatenate([user_emb[:, H:], item_emb[:, H:]], axis=0)
  xraw = jnp.concatenate([raw_lo, raw_hi], axis=0)

  side_full = jnp.concatenate(
      [gender_emb, age_emb, cat_emb, jnp.zeros((3, EMB), f32)], axis=0)
  side = jnp.concatenate([side_full[:, :H], side_full[:, H:]], axis=0)

  bidx = jnp.concatenate(
      [users, N_USERS + pos_items, N_USERS + neg_items]).astype(i32)

  outsum, rawg = _sc_prop(xraw, side, colp, rowp, valp,
                          user_gender, user_age_bucket, item_cat, bidx)

  loss, bpr = pl.pallas_call(
      _loss_body,
      out_shape=[jax.ShapeDtypeStruct((1, 1), f32)] * 2,
  )(outsum, rawg)
  return (loss[0, 0], bpr[0, 0])


# speedup vs baseline: 1.6620x; 1.0001x over previous
import functools

import jax
import jax.numpy as jnp
from jax import lax
from jax.experimental import pallas as pl
from jax.experimental.pallas import tpu as pltpu
from jax.experimental.pallas import tpu_sc as plsc

N_USERS = 25000
N_ITEMS = 25000
N_NODES = N_USERS + N_ITEMS
EMB = 64
H = 32
E = 800000
BATCH = 4096
NB = 3 * BATCH
N_LAYERS = 3
DECAY = 1e-4

NC = 2
NS = 16
CHUNK = 128
CPB = 4
TILE_CHUNKS = 392
TILE_E = TILE_CHUNKS * CHUNK
E_PAD = NS * TILE_E
NROW2D = E_PAD // CHUNK
NBLK = TILE_CHUNKS // CPB
NPAIR = NBLK // 2

SIDE = 216
ROWS_PT = N_NODES // NS
SETUP_CHUNKS = 196

_mesh = plsc.VectorSubcoreMesh(
    core_axis_name="c", subcore_axis_name="s", num_cores=NC, num_subcores=NS)


def _sc_body(xraw, side, col2d, row2d, val2d, gidx, aidx, cidx, bidx,
             outsum, rawg,
             xflat, acc, stab, i1, i2, colv, rowv, valv, gbuf,
             semi, semg, sems, semd):
  c = lax.axis_index("c")
  s = lax.axis_index("s")
  half = c * (4 * N_NODES)
  rawb = c * N_NODES
  sideb = c * SIDE

  xraw_c = xraw.at[pl.ds(rawb, N_NODES)]
  z16 = jnp.zeros((16,), jnp.float32)

  def _zero_chunk(k):
    @plsc.parallel_loop(0, CHUNK)
    def _(r, k=k):
      gbuf[k, r, pl.ds(0, 16)] = z16
      gbuf[k, r, pl.ds(16, 16)] = z16

  def _fire_acc_zero():
    def body(j, _):
      r0 = s * ROWS_PT + jnp.minimum(j * CHUNK, ROWS_PT - CHUNK)
      pltpu.async_copy(gbuf.at[2], acc.at[pl.ds(r0, CHUNK)], semd.at[2])
      return 0
    lax.fori_loop(0, 25, body, 0)

  def _wait_acc_zero():
    def body(j, _):
      pltpu.make_async_copy(gbuf.at[2], acc.at[pl.ds(0, CHUNK)],
                            semd.at[2]).wait()
      return 0
    lax.fori_loop(0, 25, body, 0)

  _scope_setup = jax.named_scope("ph_setup"); _scope_setup.__enter__()
  pltpu.sync_copy(side.at[pl.ds(sideb, 13)], stab)
  cat_c = side.at[pl.ds(sideb + 13, 200)]
  _zero_chunk(2)
  _fire_acc_zero()

  def _setup_loop(base, total_rows, add_rows, combine):
    def body(j, _):
      cid = s + NS * j

      @pl.when(cid < SETUP_CHUNKS)
      def _():
        b = jnp.minimum(cid * CHUNK, total_rows - CHUNK)

        @pl.when(j > 0)
        def _():
          pltpu.make_async_copy(gbuf.at[0], xflat.at[pl.ds(0, CHUNK)],
                                semd.at[0]).wait()
        pltpu.async_copy(xraw_c.at[pl.ds(base + b, CHUNK)], gbuf.at[0],
                         semg.at[0])
        add_rows(b)
        pltpu.make_async_copy(xraw_c.at[pl.ds(0, CHUNK)], gbuf.at[0],
                              semg.at[0]).wait()
        combine()
        pltpu.async_copy(gbuf.at[0], xflat.at[pl.ds(half + base + b, CHUNK)],
                         semd.at[0])
      return 0

    lax.fori_loop(0, 13, body, 0)
    pltpu.make_async_copy(gbuf.at[0], xflat.at[pl.ds(0, CHUNK)],
                          semd.at[0]).wait()

  def _user_idx(b):
    pltpu.async_copy(gidx.at[pl.ds(b, CHUNK)], i1, semi.at[0])
    pltpu.async_copy(aidx.at[pl.ds(b, CHUNK)], i2, semi.at[1])
    pltpu.make_async_copy(gidx.at[pl.ds(0, CHUNK)], i1, semi.at[0]).wait()
    pltpu.make_async_copy(aidx.at[pl.ds(0, CHUNK)], i2, semi.at[1]).wait()

  def _combine_user():
    @plsc.parallel_loop(0, CHUNK // 16)
    def _(g):
      gv = i1[pl.ds(g * 16, 16)]
      av = i2[pl.ds(g * 16, 16)]
      for i16 in range(16):
        e = g * 16 + i16
        gi = gv[i16]
        ai = av[i16] + 3
        for h in range(2):
          sl = pl.ds(h * 16, 16)
          gbuf[0, e, sl] = gbuf[0, e, sl] + stab[gi, sl] + stab[ai, sl]

  def _item_idx(b):
    pltpu.async_copy(cidx.at[pl.ds(b, CHUNK)], i1, semi.at[0])
    pltpu.make_async_copy(cidx.at[pl.ds(0, CHUNK)], i1, semi.at[0]).wait()
    pltpu.async_copy(cat_c.at[i1], gbuf.at[3], semg.at[2])

  def _combine_item():
    pltpu.make_async_copy(cat_c.at[i1], gbuf.at[3], semg.at[2]).wait()

    @plsc.parallel_loop(0, CHUNK)
    def _(r):
      for h in range(2):
        sl = pl.ds(h * 16, 16)
        gbuf[0, r, sl] = gbuf[0, r, sl] + gbuf[3, r, sl]

  _setup_loop(0, N_USERS, _user_idx, _combine_user)
  _setup_loop(N_USERS, N_ITEMS, _item_idx, _combine_item)

  _wait_acc_zero()
  plsc.subcore_barrier()
  _scope_setup.__exit__(None, None, None)

  def _fire_idx(buf, b):
    cr = s * TILE_CHUNKS + b * CPB
    pltpu.async_copy(col2d.at[pl.ds(cr, CPB)], colv.at[buf], semi.at[buf])
    pltpu.async_copy(row2d.at[pl.ds(cr, CPB)], rowv.at[buf], semi.at[buf])
    pltpu.async_copy(val2d.at[pl.ds(cr, CPB)], valv.at[buf], semi.at[buf])

  def _wait_idx(buf, b):
    cr = s * TILE_CHUNKS + b * CPB
    pltpu.make_async_copy(col2d.at[pl.ds(cr, CPB)], colv.at[buf],
                          semi.at[buf]).wait()
    pltpu.make_async_copy(row2d.at[pl.ds(cr, CPB)], rowv.at[buf],
                          semi.at[buf]).wait()
    pltpu.make_async_copy(val2d.at[pl.ds(cr, CPB)], valv.at[buf],
                          semi.at[buf]).wait()

  for l in range(N_LAYERS):
    _scope_l = jax.named_scope(f"ph_layer{l}"); _scope_l.__enter__()
    src = half + l * N_NODES
    dst = half + (l + 1) * N_NODES
    xsrc = xflat.at[pl.ds(src, N_NODES)]

    _fire_idx(0, 0)

    def pair_body(i, _, xsrc=xsrc):
      for p in range(2):
        b = 2 * i + p
        q = 1 - p
        _wait_idx(p, b)
        for j in range(CPB):
          @pl.when(b > 0)
          def _(j=j, q=q):
            pltpu.make_async_copy(gbuf.at[j], acc.at[rowv.at[q, j]],
                                  sems.at[j]).wait()
          pltpu.async_copy(xsrc.at[colv.at[p, j, pl.ds(0, 64)]],
                           gbuf.at[j].at[pl.ds(0, 64)], semg.at[j])
          pltpu.async_copy(xsrc.at[colv.at[p, j, pl.ds(64, 64)]],
                           gbuf.at[j].at[pl.ds(64, 64)], semg.at[j])
        @pl.when(b + 1 < NBLK)
        def _(q=q, b=b):
          _fire_idx(q, b + 1)
        for j in range(CPB):
          pltpu.make_async_copy(xsrc.at[colv.at[p, j, pl.ds(0, 64)]],
                                gbuf.at[j].at[pl.ds(0, 64)], semg.at[j]).wait()
          pltpu.make_async_copy(xsrc.at[colv.at[p, j, pl.ds(64, 64)]],
                                gbuf.at[j].at[pl.ds(64, 64)], semg.at[j]).wait()

          @plsc.parallel_loop(0, CHUNK // 16)
          def _(g, j=j, p=p):
            vv = valv[p, j, pl.ds(g * 16, 16)]
            for i16 in range(16):
              e = g * 16 + i16
              v = vv[i16]
              for h in range(2):
                sl = pl.ds(h * 16, 16)
                gbuf[j, e, sl] = gbuf[j, e, sl] * v

          pltpu.async_copy(gbuf.at[j], acc.at[rowv.at[p, j]], sems.at[j],
                           add=True)
      return 0

    lax.fori_loop(0, NPAIR, pair_body, 0)
    for j in range(CPB):
      pltpu.make_async_copy(gbuf.at[j], acc.at[rowv.at[1, j]],
                            sems.at[j]).wait()
    plsc.subcore_barrier()
    def dump_fire(j, _, dst=dst):
      r0 = s * ROWS_PT + jnp.minimum(j * CHUNK, ROWS_PT - CHUNK)
      pltpu.async_copy(acc.at[pl.ds(r0, CHUNK)],
                       xflat.at[pl.ds(dst + r0, CHUNK)], semd.at[2])
      return 0

    def dump_wait(j, _):
      pltpu.make_async_copy(acc.at[pl.ds(0, CHUNK)], xflat.at[pl.ds(0, CHUNK)],
                            semd.at[2]).wait()
      return 0

    lax.fori_loop(0, 25, dump_fire, 0)
    _zero_chunk(2)
    lax.fori_loop(0, 25, dump_wait, 0)
    _fire_acc_zero()
    _wait_acc_zero()
    plsc.subcore_barrier()
    _scope_l.__exit__(None, None, None)

  _scope_f = jax.named_scope("ph_final"); _scope_f.__enter__()
  ob = c * NB

  def fbody(j, _):
    cid = s * 6 + j
    bb = cid * CHUNK
    pltpu.async_copy(bidx.at[pl.ds(bb, CHUNK)], i1, semi.at[0])
    pltpu.make_async_copy(bidx.at[pl.ds(0, CHUNK)], i1, semi.at[0]).wait()

    @pl.when(j > 0)
    def _():
      pltpu.make_async_copy(gbuf.at[0], outsum.at[pl.ds(0, CHUNK)],
                            semd.at[0]).wait()
      pltpu.make_async_copy(gbuf.at[1], outsum.at[pl.ds(0, CHUNK)],
                            semd.at[1]).wait()

    for l in range(N_LAYERS + 1):
      xl = xflat.at[pl.ds(half + l * N_NODES, N_NODES)]
      pltpu.async_copy(xl.at[i1], gbuf.at[l], semg.at[l])
    for l in range(N_LAYERS + 1):
      pltpu.make_async_copy(xflat.at[pl.ds(0, N_NODES)].at[i1], gbuf.at[l],
                            semg.at[l]).wait()

    @plsc.parallel_loop(0, CHUNK)
    def _(r):
      for h in range(2):
        sl = pl.ds(h * 16, 16)
        gbuf[0, r, sl] = ((gbuf[0, r, sl] + gbuf[1, r, sl]) +
                          (gbuf[2, r, sl] + gbuf[3, r, sl]))

    pltpu.async_copy(xraw_c.at[i1], gbuf.at[1], semg.at[1])
    pltpu.async_copy(gbuf.at[0], outsum.at[pl.ds(ob + bb, CHUNK)], semd.at[0])
    pltpu.make_async_copy(xraw_c.at[i1], gbuf.at[1], semg.at[1]).wait()
    pltpu.async_copy(gbuf.at[1], rawg.at[pl.ds(ob + bb, CHUNK)], semd.at[1])
    return 0

  lax.fori_loop(0, 6, fbody, 0)
  for eb in range(2):
    pltpu.make_async_copy(gbuf.at[eb], outsum.at[pl.ds(0, CHUNK)],
                          semd.at[eb]).wait()
    _scope_f.__exit__(None, None, None)


_sc_prop = functools.partial(
    pl.kernel,
    out_type=[
        jax.ShapeDtypeStruct((NC * NB, H), jnp.float32),
        jax.ShapeDtypeStruct((NC * NB, H), jnp.float32),
    ],
    mesh=_mesh,
    compiler_params=pltpu.CompilerParams(use_tc_tiling_on_sc=False),
    scratch_types=[
        pltpu.HBM((NC * 4 * N_NODES, H), jnp.float32),
        pltpu.VMEM_SHARED((N_NODES, H), jnp.float32),
        pltpu.VMEM((13, H), jnp.float32),
        pltpu.VMEM((CHUNK,), jnp.int32),
        pltpu.VMEM((CHUNK,), jnp.int32),
        pltpu.VMEM((2, CPB, CHUNK), jnp.int32),
        pltpu.VMEM((2, CPB, CHUNK), jnp.int32),
        pltpu.VMEM((2, CPB, CHUNK), jnp.float32),
        pltpu.VMEM((CPB, CHUNK, H), jnp.float32),
        pltpu.SemaphoreType.DMA((2,)),
        pltpu.SemaphoreType.DMA((CPB,)),
        pltpu.SemaphoreType.DMA((CPB,)),
        pltpu.SemaphoreType.DMA((3,)),
    ],
)(_sc_body)


def _loss_body(out_ref, raw_ref, loss_ref, bpr_ref):
  o = out_ref[...]
  pos = jnp.zeros((BATCH,), jnp.float32)
  neg = jnp.zeros((BATCH,), jnp.float32)
  for hb in range(2):
    u = o[hb * NB:hb * NB + BATCH]
    p = o[hb * NB + BATCH:hb * NB + 2 * BATCH]
    n = o[hb * NB + 2 * BATCH:hb * NB + 3 * BATCH]
    pos = pos + jnp.sum(u * p, axis=1)
    neg = neg + jnp.sum(u * n, axis=1)
  diff = (pos - neg) * (1.0 / 16.0)
  bpr = -jnp.mean(jax.nn.log_sigmoid(diff))
  raw = raw_ref[...]
  reg = jnp.sum(raw * raw) * (1.0 / BATCH)
  loss_ref[...] = jnp.reshape(bpr + DECAY * reg, (1, 1))
  bpr_ref[...] = jnp.reshape(bpr, (1, 1))


def kernel(edge_index, edge_values, user_gender, user_age_bucket, item_cat,
           users, pos_items, neg_items,
           user_emb, item_emb, gender_emb, age_emb, cat_emb):
  f32, i32 = jnp.float32, jnp.int32
  pad = E_PAD - E
  rowp = jnp.concatenate([edge_index[0], jnp.zeros((pad,), i32)])
  colp = jnp.concatenate([edge_index[1], jnp.zeros((pad,), i32)])
  valp = jnp.concatenate([edge_values, jnp.zeros((pad,), f32)])
  rowp = rowp.reshape(NROW2D, CHUNK)
  colp = colp.reshape(NROW2D, CHUNK)
  valp = valp.reshape(NROW2D, CHUNK)

  raw_lo = jnp.concatenate([user_emb[:, :H], item_emb[:, :H]], axis=0)
  raw_hi = jnp.concatenate([user_emb[:, H:], item_emb[:, H:]], axis=0)
  xraw = jnp.concatenate([raw_lo, raw_hi], axis=0)

  side_full = jnp.concatenate(
      [gender_emb, age_emb, cat_emb, jnp.zeros((3, EMB), f32)], axis=0)
  side = jnp.concatenate([side_full[:, :H], side_full[:, H:]], axis=0)

  bidx = jnp.concatenate(
      [users, N_USERS + pos_items, N_USERS + neg_items]).astype(i32)

  outsum, rawg = _sc_prop(xraw, side, colp, rowp, valp,
                          user_gender, user_age_bucket, item_cat, bidx)

  loss, bpr = pl.pallas_call(
      _loss_body,
      out_shape=[jax.ShapeDtypeStruct((1, 1), f32)] * 2,
  )(outsum, rawg)
  return (loss[0, 0], bpr[0, 0])
